# baseline (speedup 1.0000x reference)
import functools

import jax
import jax.numpy as jnp
from jax import lax
from jax.experimental import pallas as pl
from jax.experimental.pallas import tpu as pltpu
from jax.experimental.pallas import tpu_sc as plsc

N = 10000
E = 320000
F_IN = 128
H = 64
C = 10
NG = 64
CP = 16

NC = 2
NS = 16
NW = NC * NS
K = 128
G = 4
CH = 80
EPAD = NW * CH * K
R = CH // G
NP = 10240
TRASH = N
RPS = NP // NS
NZ = RPS // K


@functools.partial(
    pl.kernel,
    out_type=jax.ShapeDtypeStruct((NC * NP, H), jnp.float32),
    mesh=plsc.VectorSubcoreMesh(core_axis_name="c", subcore_axis_name="s"),
    compiler_params=pltpu.CompilerParams(use_tc_tiling_on_sc=False),
    scratch_types=[
        pltpu.VMEM((CH, K), jnp.int32),
        pltpu.VMEM((CH, K), jnp.int32),
        [pltpu.VMEM((K,), jnp.int32) for _ in range(G)],
        [pltpu.VMEM((K,), jnp.int32) for _ in range(G)],
        pltpu.VMEM((G, K, H), jnp.float32),
        pltpu.VMEM((K, H), jnp.float32),
        pltpu.VMEM_SHARED((NP, H), jnp.float32),
        pltpu.SemaphoreType.DMA,
        pltpu.SemaphoreType.DMA,
        pltpu.SemaphoreType.DMA,
    ],
)
def _sc_segsum(tab_hbm, src_hbm, idx_hbm, zblk_hbm, out_hbm,
               srcbuf, idxbuf, srcc, idxc, rows, zbuf, acc,
               semA, semB, semZ):
    c = lax.axis_index("c")
    s = lax.axis_index("s")
    wid = s * NC + c

    hs = pltpu.async_copy(src_hbm.at[pl.ds(wid * CH, CH)], srcbuf, semZ)
    hi = pltpu.async_copy(idx_hbm.at[pl.ds(wid * CH, CH)], idxbuf, semZ)
    pltpu.sync_copy(zblk_hbm, zbuf)

    def zbody(k, _):
        pltpu.sync_copy(zbuf, acc.at[pl.ds(s * RPS + k * K, K)])
        return 0
    lax.fori_loop(0, NZ, zbody, 0)
    hs.wait()
    hi.wait()
    plsc.subcore_barrier()

    def round_body(q, _):
        for b in range(G):
            j = q * G + b
            for i in range(K // 16):
                srcc[b][pl.ds(16 * i, 16)] = srcbuf[j, pl.ds(16 * i, 16)]
                idxc[b][pl.ds(16 * i, 16)] = idxbuf[j, pl.ds(16 * i, 16)]
        handles = [pltpu.async_copy(tab_hbm.at[srcc[b]], rows.at[b], semA)
                   for b in range(G)]
        for b in range(G):
            handles[b].wait()
        for b in range(G):
            pltpu.sync_copy(rows.at[b], acc.at[idxc[b]], add=True)
        return 0

    lax.fori_loop(0, R, round_body, 0)
    plsc.subcore_barrier()

    def xbody(k, _):
        r = s * RPS + k * K
        pltpu.sync_copy(acc.at[pl.ds(r, K)], zbuf)
        pltpu.sync_copy(zbuf, out_hbm.at[pl.ds(c * NP + r, K)])
        return 0
    lax.fori_loop(0, NZ, xbody, 0)


def _bn_relu(z, g, be):
    m = jnp.mean(z, axis=0, keepdims=True)
    v = jnp.mean((z - m) ** 2, axis=0, keepdims=True)
    return jnp.maximum(g * (z - m) * lax.rsqrt(v + 1e-5) + be, 0.0)


def _segmax(p, bat, pool_ref):
    def gbody(g, _):
        mm = jnp.max(jnp.where(bat == g, p, -jnp.inf), axis=0, keepdims=True)
        pool_ref[pl.ds(g, 1), :] = mm
        return 0
    lax.fori_loop(0, NG, gbody, 0)


def _stage0_body(x_ref, w_ref, b_ref, g_ref, be_ref, wl_ref, bl_ref, bat_ref,
                 th_ref, h_ref, tab_ref, pool_ref):
    z = jnp.dot(x_ref[...], w_ref[...], preferred_element_type=jnp.float32)
    h = _bn_relu(z + b_ref[...], g_ref[...], be_ref[...])
    h_ref[...] = h
    tab_ref[...] = th_ref[0] * h
    p = jnp.dot(h, wl_ref[...], preferred_element_type=jnp.float32) + bl_ref[...]
    _segmax(p, bat_ref[...], pool_ref)


def _stage1_body(h_ref, acc_ref, w_ref, b_ref, g_ref, be_ref, wl_ref, bl_ref,
                 bat_ref, pin_ref, th_ref, h1_ref, tab_ref, pool_ref):
    agg = acc_ref[pl.ds(0, N), :] + acc_ref[pl.ds(NP, N), :]
    z = jnp.dot(h_ref[...] + agg, w_ref[...], preferred_element_type=jnp.float32)
    h = _bn_relu(z + b_ref[...], g_ref[...], be_ref[...])
    h1_ref[...] = h
    tab_ref[pl.ds(0, N), :] = th_ref[0] * h
    tab_ref[pl.ds(N, N), :] = th_ref[1] * h
    p = jnp.dot(h, wl_ref[...], preferred_element_type=jnp.float32) + bl_ref[...]
    _segmax(p, bat_ref[...], pool_ref)
    pool_ref[...] = pool_ref[...] + pin_ref[...]


def _stage2_body(h_ref, acc_ref, w_ref, b_ref, g_ref, be_ref, wl_ref, bl_ref,
                 bat_ref, pin_ref, pool_ref):
    agg = acc_ref[pl.ds(0, N), :] + acc_ref[pl.ds(NP, N), :]
    z = jnp.dot(h_ref[...] + agg, w_ref[...], preferred_element_type=jnp.float32)
    h = _bn_relu(z + b_ref[...], g_ref[...], be_ref[...])
    p = jnp.dot(h, wl_ref[...], preferred_element_type=jnp.float32) + bl_ref[...]
    _segmax(p, bat_ref[...], pool_ref)
    pool_ref[...] = pool_ref[...] + pin_ref[...]


def _vspec(n):
    return [pl.BlockSpec(memory_space=pltpu.VMEM) for _ in range(n)]


_SMEM = pl.BlockSpec(memory_space=pltpu.SMEM)


def kernel(x, edge_index, edge_weights, batch,
           W0, b0, g0, be0, Wl0, bl0,
           theta1, W1, b1, g1, be1, Wl1, bl1,
           theta2, W2, b2, g2, be2, Wl2, bl2):
    src = edge_index[0]
    dst = edge_index[1]
    ew = edge_weights

    zpad = jnp.zeros((EPAD - E,), jnp.int32)
    tpad = jnp.full((EPAD - E,), TRASH, jnp.int32)
    gsrc1 = jnp.concatenate([jnp.where(ew <= 0, src, 0), zpad]).reshape(NW * CH, K)
    sidx1 = jnp.concatenate([jnp.where(ew <= 0, dst, TRASH), tpad]).reshape(NW * CH, K)
    gsrc2 = jnp.concatenate([jnp.where(ew <= 1, ew * N + src, 0), zpad]).reshape(NW * CH, K)
    sidx2 = jnp.concatenate([jnp.where(ew <= 1, dst, TRASH), tpad]).reshape(NW * CH, K)
    zblk = jnp.zeros((K, H), jnp.float32)

    def row(v):
        return v.reshape(1, -1)
    def padwl(wl):
        return jnp.pad(wl, ((0, 0), (0, CP - C)))
    def padbl(bl):
        return jnp.pad(bl, (0, CP - C)).reshape(1, CP)
    bat = batch.reshape(N, 1)

    h0, tab0, p0 = pl.pallas_call(
        _stage0_body,
        in_specs=_vspec(8) + [_SMEM],
        out_shape=(jax.ShapeDtypeStruct((N, H), jnp.float32),
                   jax.ShapeDtypeStruct((N, H), jnp.float32),
                   jax.ShapeDtypeStruct((NG, CP), jnp.float32)),
    )(x, W0, row(b0), row(g0), row(be0), padwl(Wl0), padbl(bl0), bat, theta1)

    acc1 = _sc_segsum(tab0, gsrc1, sidx1, zblk)

    h1, tab1, p1 = pl.pallas_call(
        _stage1_body,
        in_specs=_vspec(10) + [_SMEM],
        out_shape=(jax.ShapeDtypeStruct((N, H), jnp.float32),
                   jax.ShapeDtypeStruct((2 * N, H), jnp.float32),
                   jax.ShapeDtypeStruct((NG, CP), jnp.float32)),
    )(h0, acc1, W1, row(b1), row(g1), row(be1), padwl(Wl1), padbl(bl1),
      bat, p0, theta2)

    acc2 = _sc_segsum(tab1, gsrc2, sidx2, zblk)

    p2, = pl.pallas_call(
        _stage2_body,
        in_specs=_vspec(10),
        out_shape=(jax.ShapeDtypeStruct((NG, CP), jnp.float32),),
    )(h1, acc2, W2, row(b2), row(g2), row(be2), padwl(Wl2), padbl(bl2),
      bat, p1)

    return p2[:, :C]


# speedup vs baseline: 13.4568x; 13.4568x over previous
import functools

import jax
import jax.numpy as jnp
from jax import lax
from jax.experimental import pallas as pl
from jax.experimental.pallas import tpu as pltpu
from jax.experimental.pallas import tpu_sc as plsc

N = 10000
E = 320000
F_IN = 128
H = 64
C = 10
NG = 64
CP = 16

NC = 2
NS = 16
NW = NC * NS
K = 128
G = 4
CH = 80
EPAD = NW * CH * K
R = CH // G
NP = 10240
TRASH = N
RPS = NP // NS
NZ = RPS // K


@functools.partial(
    pl.kernel,
    out_type=jax.ShapeDtypeStruct((NC * NP, H), jnp.float32),
    mesh=plsc.VectorSubcoreMesh(core_axis_name="c", subcore_axis_name="s"),
    compiler_params=pltpu.CompilerParams(use_tc_tiling_on_sc=False),
    scratch_types=[
        pltpu.VMEM((CH, K), jnp.int32),
        pltpu.VMEM((CH, K), jnp.int32),
        [pltpu.VMEM((K,), jnp.int32) for _ in range(G)],
        [pltpu.VMEM((K,), jnp.int32) for _ in range(G)],
        pltpu.VMEM((G, K, H), jnp.float32),
        pltpu.VMEM((K, H), jnp.float32),
        pltpu.VMEM_SHARED((NP, H), jnp.float32),
        pltpu.SemaphoreType.DMA,
        pltpu.SemaphoreType.DMA,
        pltpu.SemaphoreType.DMA,
    ],
)
def _sc_segsum(tab_hbm, src_hbm, idx_hbm, zblk_hbm, out_hbm,
               srcbuf, idxbuf, srcc, idxc, rows, zbuf, acc,
               semA, semB, semZ):
    c = lax.axis_index("c")
    s = lax.axis_index("s")
    wid = s * NC + c

    hs = pltpu.async_copy(src_hbm.at[pl.ds(wid * CH, CH)], srcbuf, semZ)
    hi = pltpu.async_copy(idx_hbm.at[pl.ds(wid * CH, CH)], idxbuf, semZ)
    pltpu.sync_copy(zblk_hbm, zbuf)

    def zbody(k, _):
        pltpu.sync_copy(zbuf, acc.at[pl.ds(s * RPS + k * K, K)])
        return 0
    lax.fori_loop(0, NZ, zbody, 0)
    hs.wait()
    hi.wait()
    plsc.subcore_barrier()

    def round_body(q, _):
        for b in range(G):
            j = q * G + b
            for i in range(K // 16):
                srcc[b][pl.ds(16 * i, 16)] = srcbuf[j, pl.ds(16 * i, 16)]
                idxc[b][pl.ds(16 * i, 16)] = idxbuf[j, pl.ds(16 * i, 16)]
        handles = [pltpu.async_copy(tab_hbm.at[srcc[b]], rows.at[b], semA)
                   for b in range(G)]
        for b in range(G):
            handles[b].wait()
        for b in range(G):
            pltpu.sync_copy(rows.at[b], acc.at[idxc[b]], add=True)
        return 0

    lax.fori_loop(0, R, round_body, 0)
    plsc.subcore_barrier()

    def xbody(k, _):
        r = s * RPS + k * K
        pltpu.sync_copy(acc.at[pl.ds(r, K)], zbuf)
        pltpu.sync_copy(zbuf, out_hbm.at[pl.ds(c * NP + r, K)])
        return 0
    lax.fori_loop(0, NZ, xbody, 0)


def _bn_relu(z, g, be):
    m = jnp.mean(z, axis=0, keepdims=True)
    v = jnp.mean((z - m) ** 2, axis=0, keepdims=True)
    return jnp.maximum(g * (z - m) * lax.rsqrt(v + 1e-5) + be, 0.0)


def _segmax(p, bat, pool_ref):
    def gbody(g, _):
        mm = jnp.max(jnp.where(bat == g, p, -jnp.inf), axis=0, keepdims=True)
        pool_ref[pl.ds(g, 1), :] = mm
        return 0
    lax.fori_loop(0, NG, gbody, 0)


def _stage0_body(x_ref, w_ref, b_ref, g_ref, be_ref, wl_ref, bl_ref, bat_ref,
                 th_ref, h_ref, tab_ref, pool_ref):
    z = jnp.dot(x_ref[...], w_ref[...], preferred_element_type=jnp.float32)
    h = _bn_relu(z + b_ref[...], g_ref[...], be_ref[...])
    h_ref[...] = h
    tab_ref[...] = th_ref[0] * h
    p = jnp.dot(h, wl_ref[...], preferred_element_type=jnp.float32) + bl_ref[...]
    _segmax(p, bat_ref[...], pool_ref)


def _stage1_body(h_ref, acc_ref, w_ref, b_ref, g_ref, be_ref, wl_ref, bl_ref,
                 bat_ref, pin_ref, th_ref, h1_ref, tab_ref, pool_ref):
    agg = acc_ref[pl.ds(0, N), :] + acc_ref[pl.ds(NP, N), :]
    z = jnp.dot(h_ref[...] + agg, w_ref[...], preferred_element_type=jnp.float32)
    h = _bn_relu(z + b_ref[...], g_ref[...], be_ref[...])
    h1_ref[...] = h
    tab_ref[pl.ds(0, N), :] = th_ref[0] * h
    tab_ref[pl.ds(N, N), :] = th_ref[1] * h
    p = jnp.dot(h, wl_ref[...], preferred_element_type=jnp.float32) + bl_ref[...]
    _segmax(p, bat_ref[...], pool_ref)
    pool_ref[...] = pool_ref[...] + pin_ref[...]


def _stage2_body(h_ref, acc_ref, w_ref, b_ref, g_ref, be_ref, wl_ref, bl_ref,
                 bat_ref, pin_ref, pool_ref):
    agg = acc_ref[pl.ds(0, N), :] + acc_ref[pl.ds(NP, N), :]
    z = jnp.dot(h_ref[...] + agg, w_ref[...], preferred_element_type=jnp.float32)
    h = _bn_relu(z + b_ref[...], g_ref[...], be_ref[...])
    p = jnp.dot(h, wl_ref[...], preferred_element_type=jnp.float32) + bl_ref[...]
    _segmax(p, bat_ref[...], pool_ref)
    pool_ref[...] = pool_ref[...] + pin_ref[...]


def _vspec(n):
    return [pl.BlockSpec(memory_space=pltpu.VMEM) for _ in range(n)]


_SMEM = pl.BlockSpec(memory_space=pltpu.SMEM)


def kernel(x, edge_index, edge_weights, batch,
           W0, b0, g0, be0, Wl0, bl0,
           theta1, W1, b1, g1, be1, Wl1, bl1,
           theta2, W2, b2, g2, be2, Wl2, bl2):
    src = edge_index[0]
    dst = edge_index[1]
    ew = edge_weights

    spread = (jnp.arange(EPAD - E, dtype=jnp.int32) * 37) % N
    tpad = jnp.full((EPAD - E,), TRASH, jnp.int32)
    gsrc1 = jnp.concatenate([src, spread]).reshape(NW * CH, K)
    sidx1 = jnp.concatenate([jnp.where(ew <= 0, dst, TRASH), tpad]).reshape(NW * CH, K)
    gsrc2 = jnp.concatenate([jnp.where(ew <= 1, ew * N + src, src), spread]).reshape(NW * CH, K)
    sidx2 = jnp.concatenate([jnp.where(ew <= 1, dst, TRASH), tpad]).reshape(NW * CH, K)
    zblk = jnp.zeros((K, H), jnp.float32)

    def row(v):
        return v.reshape(1, -1)
    def padwl(wl):
        return jnp.pad(wl, ((0, 0), (0, CP - C)))
    def padbl(bl):
        return jnp.pad(bl, (0, CP - C)).reshape(1, CP)
    bat = batch.reshape(N, 1)

    h0, tab0, p0 = pl.pallas_call(
        _stage0_body,
        in_specs=_vspec(8) + [_SMEM],
        out_shape=(jax.ShapeDtypeStruct((N, H), jnp.float32),
                   jax.ShapeDtypeStruct((N, H), jnp.float32),
                   jax.ShapeDtypeStruct((NG, CP), jnp.float32)),
    )(x, W0, row(b0), row(g0), row(be0), padwl(Wl0), padbl(bl0), bat, theta1)

    acc1 = _sc_segsum(tab0, gsrc1, sidx1, zblk)

    h1, tab1, p1 = pl.pallas_call(
        _stage1_body,
        in_specs=_vspec(10) + [_SMEM],
        out_shape=(jax.ShapeDtypeStruct((N, H), jnp.float32),
                   jax.ShapeDtypeStruct((2 * N, H), jnp.float32),
                   jax.ShapeDtypeStruct((NG, CP), jnp.float32)),
    )(h0, acc1, W1, row(b1), row(g1), row(be1), padwl(Wl1), padbl(bl1),
      bat, p0, theta2)

    acc2 = _sc_segsum(tab1, gsrc2, sidx2, zblk)

    p2, = pl.pallas_call(
        _stage2_body,
        in_specs=_vspec(10),
        out_shape=(jax.ShapeDtypeStruct((NG, CP), jnp.float32),),
    )(h1, acc2, W2, row(b2), row(g2), row(be2), padwl(Wl2), padbl(bl2),
      bat, p1)

    return p2[:, :C]


# speedup vs baseline: 16.8282x; 1.2505x over previous
import functools

import jax
import jax.numpy as jnp
from jax import lax
from jax.experimental import pallas as pl
from jax.experimental.pallas import tpu as pltpu
from jax.experimental.pallas import tpu_sc as plsc

N = 10000
E = 320000
F_IN = 128
H = 64
C = 10
NG = 64
CP = 16

NC = 2
NS = 16
NW = NC * NS
K = 128
G = 4
CH = 80
EPAD = NW * CH * K
R = CH // G
NP = 10240
TRASH = N
RPS = NP // NS
NZ = RPS // K


@functools.partial(
    pl.kernel,
    out_type=jax.ShapeDtypeStruct((NC * NP, H), jnp.float32),
    mesh=plsc.VectorSubcoreMesh(core_axis_name="c", subcore_axis_name="s"),
    compiler_params=pltpu.CompilerParams(use_tc_tiling_on_sc=False),
    scratch_types=[
        pltpu.VMEM((CH, K), jnp.int32),
        pltpu.VMEM((CH, K), jnp.int32),
        [pltpu.VMEM((K,), jnp.int32) for _ in range(G)],
        [pltpu.VMEM((K,), jnp.int32) for _ in range(G)],
        pltpu.VMEM((G, K, H), jnp.float32),
        pltpu.VMEM((K, H), jnp.float32),
        pltpu.VMEM_SHARED((NP, H), jnp.float32),
        pltpu.SemaphoreType.DMA,
        pltpu.SemaphoreType.DMA,
        pltpu.SemaphoreType.DMA,
    ],
)
def _sc_segsum(tab_hbm, src_hbm, idx_hbm, zblk_hbm, out_hbm,
               srcbuf, idxbuf, srcc, idxc, rows, zbuf, acc,
               semA, semB, semZ):
    c = lax.axis_index("c")
    s = lax.axis_index("s")
    wid = s * NC + c

    hs = pltpu.async_copy(src_hbm.at[pl.ds(wid * CH, CH)], srcbuf, semZ)
    hi = pltpu.async_copy(idx_hbm.at[pl.ds(wid * CH, CH)], idxbuf, semZ)
    pltpu.sync_copy(zblk_hbm, zbuf)

    def zbody(k, _):
        pltpu.sync_copy(zbuf, acc.at[pl.ds(s * RPS + k * K, K)])
        return 0
    lax.fori_loop(0, NZ, zbody, 0)
    hs.wait()
    hi.wait()
    plsc.subcore_barrier()

    def round_body(q, _):
        for b in range(G):
            j = q * G + b
            for i in range(K // 16):
                srcc[b][pl.ds(16 * i, 16)] = srcbuf[j, pl.ds(16 * i, 16)]
                idxc[b][pl.ds(16 * i, 16)] = idxbuf[j, pl.ds(16 * i, 16)]
        handles = [pltpu.async_copy(tab_hbm.at[srcc[b]], rows.at[b], semA)
                   for b in range(G)]
        for b in range(G):
            handles[b].wait()
        for b in range(G):
            pltpu.sync_copy(rows.at[b], acc.at[idxc[b]], add=True)
        return 0

    lax.fori_loop(0, R, round_body, 0)
    plsc.subcore_barrier()

    def xbody(k, _):
        r = s * RPS + k * K
        pltpu.sync_copy(acc.at[pl.ds(r, K)], zbuf)
        pltpu.sync_copy(zbuf, out_hbm.at[pl.ds(c * NP + r, K)])
        return 0
    lax.fori_loop(0, NZ, xbody, 0)


def _bn_relu(z, g, be):
    m = jnp.mean(z, axis=0, keepdims=True)
    v = jnp.mean((z - m) ** 2, axis=0, keepdims=True)
    return jnp.maximum(g * (z - m) * lax.rsqrt(v + 1e-5) + be, 0.0)


def _segmax(p, bat, pool_ref):
    def gbody(g, _):
        mm = jnp.max(jnp.where(bat == g, p, -jnp.inf), axis=0, keepdims=True)
        pool_ref[pl.ds(g, 1), :] = mm
        return 0
    lax.fori_loop(0, NG, gbody, 0)


def _stage0_body(x_ref, w_ref, b_ref, g_ref, be_ref, wl_ref, bl_ref, bat_ref,
                 th_ref, h_ref, tab_ref, pool_ref):
    z = jnp.dot(x_ref[...], w_ref[...], preferred_element_type=jnp.float32)
    h = _bn_relu(z + b_ref[...], g_ref[...], be_ref[...])
    h_ref[...] = h
    tab_ref[...] = th_ref[0] * h
    p = jnp.dot(h, wl_ref[...], preferred_element_type=jnp.float32) + bl_ref[...]
    _segmax(p, bat_ref[...], pool_ref)


def _stage1_body(h_ref, acc_ref, w_ref, b_ref, g_ref, be_ref, wl_ref, bl_ref,
                 bat_ref, pin_ref, th_ref, h1_ref, tab_ref, pool_ref):
    agg = acc_ref[pl.ds(0, N), :] + acc_ref[pl.ds(NP, N), :]
    z = jnp.dot(h_ref[...] + agg, w_ref[...], preferred_element_type=jnp.float32)
    h = _bn_relu(z + b_ref[...], g_ref[...], be_ref[...])
    h1_ref[...] = h
    tab_ref[pl.ds(0, N), :] = th_ref[0] * h
    tab_ref[pl.ds(N, N), :] = th_ref[1] * h
    p = jnp.dot(h, wl_ref[...], preferred_element_type=jnp.float32) + bl_ref[...]
    _segmax(p, bat_ref[...], pool_ref)
    pool_ref[...] = pool_ref[...] + pin_ref[...]


def _stage2_body(h_ref, acc_ref, w_ref, b_ref, g_ref, be_ref, wl_ref, bl_ref,
                 bat_ref, pin_ref, pool_ref):
    agg = acc_ref[pl.ds(0, N), :] + acc_ref[pl.ds(NP, N), :]
    z = jnp.dot(h_ref[...] + agg, w_ref[...], preferred_element_type=jnp.float32)
    h = _bn_relu(z + b_ref[...], g_ref[...], be_ref[...])
    p = jnp.dot(h, wl_ref[...], preferred_element_type=jnp.float32) + bl_ref[...]
    _segmax(p, bat_ref[...], pool_ref)
    pool_ref[...] = pool_ref[...] + pin_ref[...]


def _vspec(n):
    return [pl.BlockSpec(memory_space=pltpu.VMEM) for _ in range(n)]


_SMEM = pl.BlockSpec(memory_space=pltpu.SMEM)


def kernel(x, edge_index, edge_weights, batch,
           W0, b0, g0, be0, Wl0, bl0,
           theta1, W1, b1, g1, be1, Wl1, bl1,
           theta2, W2, b2, g2, be2, Wl2, bl2):
    src = edge_index[0]
    dst = edge_index[1]
    ew = edge_weights

    spread = (jnp.arange(EPAD - E, dtype=jnp.int32) * 37) % N
    trash = TRASH + (jnp.arange(E, dtype=jnp.int32) % (NP - N))
    tpad = TRASH + ((jnp.arange(EPAD - E, dtype=jnp.int32) * 7) % (NP - N))
    gsrc1 = jnp.concatenate([src, spread]).reshape(NW * CH, K)
    sidx1 = jnp.concatenate([jnp.where(ew <= 0, dst, trash), tpad]).reshape(NW * CH, K)
    gsrc2 = jnp.concatenate([jnp.where(ew <= 1, ew * N + src, src), spread]).reshape(NW * CH, K)
    sidx2 = jnp.concatenate([jnp.where(ew <= 1, dst, trash), tpad]).reshape(NW * CH, K)
    zblk = jnp.zeros((K, H), jnp.float32)

    def row(v):
        return v.reshape(1, -1)
    def padwl(wl):
        return jnp.pad(wl, ((0, 0), (0, CP - C)))
    def padbl(bl):
        return jnp.pad(bl, (0, CP - C)).reshape(1, CP)
    bat = batch.reshape(N, 1)

    h0, tab0, p0 = pl.pallas_call(
        _stage0_body,
        in_specs=_vspec(8) + [_SMEM],
        out_shape=(jax.ShapeDtypeStruct((N, H), jnp.float32),
                   jax.ShapeDtypeStruct((N, H), jnp.float32),
                   jax.ShapeDtypeStruct((NG, CP), jnp.float32)),
    )(x, W0, row(b0), row(g0), row(be0), padwl(Wl0), padbl(bl0), bat, theta1)

    acc1 = _sc_segsum(tab0, gsrc1, sidx1, zblk)

    h1, tab1, p1 = pl.pallas_call(
        _stage1_body,
        in_specs=_vspec(10) + [_SMEM],
        out_shape=(jax.ShapeDtypeStruct((N, H), jnp.float32),
                   jax.ShapeDtypeStruct((2 * N, H), jnp.float32),
                   jax.ShapeDtypeStruct((NG, CP), jnp.float32)),
    )(h0, acc1, W1, row(b1), row(g1), row(be1), padwl(Wl1), padbl(bl1),
      bat, p0, theta2)

    acc2 = _sc_segsum(tab1, gsrc2, sidx2, zblk)

    p2, = pl.pallas_call(
        _stage2_body,
        in_specs=_vspec(10),
        out_shape=(jax.ShapeDtypeStruct((NG, CP), jnp.float32),),
    )(h1, acc2, W2, row(b2), row(g2), row(be2), padwl(Wl2), padbl(bl2),
      bat, p1)

    return p2[:, :C]


# speedup vs baseline: 22.2330x; 1.3212x over previous
import functools

import jax
import jax.numpy as jnp
from jax import lax
from jax.experimental import pallas as pl
from jax.experimental.pallas import tpu as pltpu
from jax.experimental.pallas import tpu_sc as plsc

N = 10000
E = 320000
F_IN = 128
H = 64
C = 10
NG = 64
CP = 16

NC = 2
NS = 16
NW = NC * NS
K = 128
G = 4
CH = 80
EPAD = NW * CH * K
R = CH // G
NP = 10240
TRASH = N
RPS = NP // NS
NZ = RPS // K


@functools.partial(
    pl.kernel,
    out_type=jax.ShapeDtypeStruct((NC * NP, H), jnp.float32),
    mesh=plsc.VectorSubcoreMesh(core_axis_name="c", subcore_axis_name="s"),
    compiler_params=pltpu.CompilerParams(use_tc_tiling_on_sc=False),
    scratch_types=[
        pltpu.VMEM((CH, K), jnp.int32),
        pltpu.VMEM((CH, K), jnp.int32),
        [pltpu.VMEM((K,), jnp.int32) for _ in range(G)],
        [pltpu.VMEM((K,), jnp.int32) for _ in range(G)],
        pltpu.VMEM((G, K, H), jnp.float32),
        pltpu.VMEM((K, H), jnp.float32),
        pltpu.VMEM_SHARED((NP, H), jnp.float32),
        pltpu.SemaphoreType.DMA,
        pltpu.SemaphoreType.DMA,
        pltpu.SemaphoreType.DMA,
    ],
)
def _sc_segsum(tab_hbm, src_hbm, idx_hbm, zblk_hbm, out_hbm,
               srcbuf, idxbuf, srcc, idxc, rows, zbuf, acc,
               semA, semB, semZ):
    c = lax.axis_index("c")
    s = lax.axis_index("s")
    wid = s * NC + c

    hs = pltpu.async_copy(src_hbm.at[pl.ds(wid * CH, CH)], srcbuf, semZ)
    hi = pltpu.async_copy(idx_hbm.at[pl.ds(wid * CH, CH)], idxbuf, semZ)
    pltpu.sync_copy(zblk_hbm, zbuf)

    def zbody(k, _):
        pltpu.sync_copy(zbuf, acc.at[pl.ds(s * RPS + k * K, K)])
        return 0
    lax.fori_loop(0, NZ, zbody, 0)
    hs.wait()
    hi.wait()
    plsc.subcore_barrier()

    def round_body(q, _):
        for b in range(G):
            j = q * G + b
            for i in range(K // 16):
                srcc[b][pl.ds(16 * i, 16)] = srcbuf[j, pl.ds(16 * i, 16)]
                idxc[b][pl.ds(16 * i, 16)] = idxbuf[j, pl.ds(16 * i, 16)]
        handles = [pltpu.async_copy(tab_hbm.at[srcc[b]], rows.at[b], semA)
                   for b in range(G)]
        for b in range(G):
            handles[b].wait()
        for b in range(G):
            pltpu.sync_copy(rows.at[b], acc.at[idxc[b]], add=True)
        return 0

    lax.fori_loop(0, R, round_body, 0)
    plsc.subcore_barrier()

    def xbody(k, _):
        r = s * RPS + k * K
        pltpu.sync_copy(acc.at[pl.ds(r, K)], zbuf)
        pltpu.sync_copy(zbuf, out_hbm.at[pl.ds(c * NP + r, K)])
        return 0
    lax.fori_loop(0, NZ, xbody, 0)


def _bn_relu(z, g, be):
    m = jnp.mean(z, axis=0, keepdims=True)
    v = jnp.mean((z - m) ** 2, axis=0, keepdims=True)
    return jnp.maximum(g * (z - m) * lax.rsqrt(v + 1e-5) + be, 0.0)


NB = N // 8


def _segmax(p, bat, bat_ref, pscr_ref, pool_ref):
    pscr_ref[...] = p
    bm = jnp.max(p.reshape(NB, 8, CP), axis=1)
    lane = lax.broadcasted_iota(jnp.int32, (1, 2 * NG), 1)
    starts = jnp.sum((bat < lane).astype(jnp.float32), axis=0,
                     keepdims=True).astype(jnp.int32)
    biota = lax.broadcasted_iota(jnp.int32, (NB, 1), 0)

    def gbody(g, _):
        s_g = jnp.sum(jnp.where(lane == g, starts, 0))
        e_g = jnp.sum(jnp.where(lane == g + 1, starts, 0))
        b0 = jnp.clip(s_g // 8, 0, NB - 1)
        b1 = jnp.clip((e_g - 1) // 8, 0, NB - 1)
        inner = jnp.max(jnp.where((biota > b0) & (biota < b1), bm, -jnp.inf),
                        axis=0, keepdims=True)
        r0 = pscr_ref[pl.ds(b0 * 8, 8), :]
        m0 = bat_ref[pl.ds(b0 * 8, 8), :] == g
        v0 = jnp.max(jnp.where(m0, r0, -jnp.inf), axis=0, keepdims=True)
        r1 = pscr_ref[pl.ds(b1 * 8, 8), :]
        m1 = bat_ref[pl.ds(b1 * 8, 8), :] == g
        v1 = jnp.max(jnp.where(m1, r1, -jnp.inf), axis=0, keepdims=True)
        pool_ref[pl.ds(g, 1), :] = jnp.maximum(jnp.maximum(inner, v0), v1)
        return 0
    lax.fori_loop(0, NG, gbody, 0)


def _stage0_body(x_ref, w_ref, b_ref, g_ref, be_ref, wl_ref, bl_ref, bat_ref,
                 th_ref, h_ref, tab_ref, pool_ref, pscr_ref):
    z = jnp.dot(x_ref[...], w_ref[...], preferred_element_type=jnp.float32)
    h = _bn_relu(z + b_ref[...], g_ref[...], be_ref[...])
    h_ref[...] = h
    tab_ref[...] = th_ref[0] * h
    p = jnp.dot(h, wl_ref[...], preferred_element_type=jnp.float32) + bl_ref[...]
    _segmax(p, bat_ref[...], bat_ref, pscr_ref, pool_ref)


def _stage1_body(h_ref, acc_ref, w_ref, b_ref, g_ref, be_ref, wl_ref, bl_ref,
                 bat_ref, pin_ref, th_ref, h1_ref, tab_ref, pool_ref, pscr_ref):
    agg = acc_ref[pl.ds(0, N), :] + acc_ref[pl.ds(NP, N), :]
    z = jnp.dot(h_ref[...] + agg, w_ref[...], preferred_element_type=jnp.float32)
    h = _bn_relu(z + b_ref[...], g_ref[...], be_ref[...])
    h1_ref[...] = h
    tab_ref[pl.ds(0, N), :] = th_ref[0] * h
    tab_ref[pl.ds(N, N), :] = th_ref[1] * h
    p = jnp.dot(h, wl_ref[...], preferred_element_type=jnp.float32) + bl_ref[...]
    _segmax(p, bat_ref[...], bat_ref, pscr_ref, pool_ref)
    pool_ref[...] = pool_ref[...] + pin_ref[...]


def _stage2_body(h_ref, acc_ref, w_ref, b_ref, g_ref, be_ref, wl_ref, bl_ref,
                 bat_ref, pin_ref, pool_ref, pscr_ref):
    agg = acc_ref[pl.ds(0, N), :] + acc_ref[pl.ds(NP, N), :]
    z = jnp.dot(h_ref[...] + agg, w_ref[...], preferred_element_type=jnp.float32)
    h = _bn_relu(z + b_ref[...], g_ref[...], be_ref[...])
    p = jnp.dot(h, wl_ref[...], preferred_element_type=jnp.float32) + bl_ref[...]
    _segmax(p, bat_ref[...], bat_ref, pscr_ref, pool_ref)
    pool_ref[...] = pool_ref[...] + pin_ref[...]


def _vspec(n):
    return [pl.BlockSpec(memory_space=pltpu.VMEM) for _ in range(n)]


_SMEM = pl.BlockSpec(memory_space=pltpu.SMEM)


def kernel(x, edge_index, edge_weights, batch,
           W0, b0, g0, be0, Wl0, bl0,
           theta1, W1, b1, g1, be1, Wl1, bl1,
           theta2, W2, b2, g2, be2, Wl2, bl2):
    src = edge_index[0]
    dst = edge_index[1]
    ew = edge_weights

    spread = (jnp.arange(EPAD - E, dtype=jnp.int32) * 37) % N
    trash = TRASH + (jnp.arange(E, dtype=jnp.int32) % (NP - N))
    tpad = TRASH + ((jnp.arange(EPAD - E, dtype=jnp.int32) * 7) % (NP - N))
    gsrc1 = jnp.concatenate([src, spread]).reshape(NW * CH, K)
    sidx1 = jnp.concatenate([jnp.where(ew <= 0, dst, trash), tpad]).reshape(NW * CH, K)
    gsrc2 = jnp.concatenate([jnp.where(ew <= 1, ew * N + src, src), spread]).reshape(NW * CH, K)
    sidx2 = jnp.concatenate([jnp.where(ew <= 1, dst, trash), tpad]).reshape(NW * CH, K)
    zblk = jnp.zeros((K, H), jnp.float32)

    def row(v):
        return v.reshape(1, -1)
    def padwl(wl):
        return jnp.pad(wl, ((0, 0), (0, CP - C)))
    def padbl(bl):
        return jnp.pad(bl, (0, CP - C)).reshape(1, CP)
    bat = batch.reshape(N, 1)

    h0, tab0, p0 = pl.pallas_call(
        _stage0_body,
        in_specs=_vspec(8) + [_SMEM],
        out_shape=(jax.ShapeDtypeStruct((N, H), jnp.float32),
                   jax.ShapeDtypeStruct((N, H), jnp.float32),
                   jax.ShapeDtypeStruct((NG, CP), jnp.float32)),
        scratch_shapes=[pltpu.VMEM((N, CP), jnp.float32)],
    )(x, W0, row(b0), row(g0), row(be0), padwl(Wl0), padbl(bl0), bat, theta1)

    acc1 = _sc_segsum(tab0, gsrc1, sidx1, zblk)

    h1, tab1, p1 = pl.pallas_call(
        _stage1_body,
        in_specs=_vspec(10) + [_SMEM],
        out_shape=(jax.ShapeDtypeStruct((N, H), jnp.float32),
                   jax.ShapeDtypeStruct((2 * N, H), jnp.float32),
                   jax.ShapeDtypeStruct((NG, CP), jnp.float32)),
        scratch_shapes=[pltpu.VMEM((N, CP), jnp.float32)],
    )(h0, acc1, W1, row(b1), row(g1), row(be1), padwl(Wl1), padbl(bl1),
      bat, p0, theta2)

    acc2 = _sc_segsum(tab1, gsrc2, sidx2, zblk)

    p2, = pl.pallas_call(
        _stage2_body,
        in_specs=_vspec(10),
        out_shape=(jax.ShapeDtypeStruct((NG, CP), jnp.float32),),
        scratch_shapes=[pltpu.VMEM((N, CP), jnp.float32)],
    )(h1, acc2, W2, row(b2), row(g2), row(be2), padwl(Wl2), padbl(bl2),
      bat, p1)

    return p2[:, :C]


# speedup vs baseline: 23.8005x; 1.0705x over previous
import functools

import jax
import jax.numpy as jnp
from jax import lax
from jax.experimental import pallas as pl
from jax.experimental.pallas import tpu as pltpu
from jax.experimental.pallas import tpu_sc as plsc

N = 10000
E = 320000
F_IN = 128
H = 64
C = 10
NG = 64
CP = 16

NC = 2
NS = 16
NW = NC * NS
K = 128
G = 4
CH = 80
EPAD = NW * CH * K
R = CH // G
NP = 10240
TRASH = N
RPS = NP // NS
NZ = RPS // K


@functools.partial(
    pl.kernel,
    out_type=jax.ShapeDtypeStruct((NC * NP, H), jnp.float32),
    mesh=plsc.VectorSubcoreMesh(core_axis_name="c", subcore_axis_name="s"),
    compiler_params=pltpu.CompilerParams(use_tc_tiling_on_sc=False),
    scratch_types=[
        pltpu.VMEM((CH, K), jnp.int32),
        pltpu.VMEM((CH, K), jnp.int32),
        [pltpu.VMEM((K,), jnp.int32) for _ in range(2 * G)],
        [pltpu.VMEM((K,), jnp.int32) for _ in range(2 * G)],
        pltpu.VMEM((2 * G, K, H), jnp.float32),
        pltpu.VMEM_SHARED((NP, H), jnp.float32),
        pltpu.SemaphoreType.DMA,
        pltpu.SemaphoreType.DMA,
        pltpu.SemaphoreType.DMA,
        pltpu.SemaphoreType.DMA,
        pltpu.SemaphoreType.DMA,
    ],
)
def _sc_segsum(tab_hbm, src_hbm, idx_hbm, zblk_hbm, out_hbm,
               srcbuf, idxbuf, srcc, idxc, rows, acc,
               semG0, semG1, semS0, semS1, semZ):
    c = lax.axis_index("c")
    s = lax.axis_index("s")
    wid = s * NC + c
    S0, S1 = (0, 1, 2, 3), (4, 5, 6, 7)

    hs = pltpu.async_copy(src_hbm.at[pl.ds(wid * CH, CH)], srcbuf, semZ)
    hi = pltpu.async_copy(idx_hbm.at[pl.ds(wid * CH, CH)], idxbuf, semZ)
    pltpu.sync_copy(zblk_hbm, rows.at[0])

    def zbody(k, _):
        pltpu.sync_copy(rows.at[0], acc.at[pl.ds(s * RPS + k * K, K)])
        return 0
    lax.fori_loop(0, NZ, zbody, 0)
    hs.wait()
    hi.wait()
    plsc.subcore_barrier()

    def fire_g(h, slots, sem):
        for b in range(G):
            j = h * G + b
            sl = slots[b]
            for i in range(K // 16):
                srcc[sl][pl.ds(16 * i, 16)] = srcbuf[j, pl.ds(16 * i, 16)]
                idxc[sl][pl.ds(16 * i, 16)] = idxbuf[j, pl.ds(16 * i, 16)]
            pltpu.async_copy(tab_hbm.at[srcc[sl]], rows.at[sl], sem)

    def drain_g(slots, sem):
        for sl in slots:
            pltpu.make_async_copy(tab_hbm.at[srcc[sl]], rows.at[sl], sem).wait()

    def fire_s(slots, sem):
        for sl in slots:
            pltpu.async_copy(rows.at[sl], acc.at[idxc[sl]], sem, add=True)

    def drain_s(slots, sem):
        for sl in slots:
            pltpu.make_async_copy(rows.at[sl], acc.at[idxc[sl]], sem).wait()

    NHB = CH // (2 * G)
    fire_g(0, S0, semG0)
    fire_g(1, S1, semG1)

    def body(u, fire_next):
        drain_g(S0, semG0)
        fire_s(S0, semS0)
        drain_g(S1, semG1)
        fire_s(S1, semS1)
        drain_s(S0, semS0)
        if fire_next:
            fire_g(2 * u + 2, S0, semG0)
        drain_s(S1, semS1)
        if fire_next:
            fire_g(2 * u + 3, S1, semG1)
        return 0

    lax.fori_loop(0, NHB - 1, lambda u, _: body(u, True), 0)
    body(NHB - 1, False)
    plsc.subcore_barrier()

    def xbody(k, _):
        r = s * RPS + k * K
        pltpu.sync_copy(acc.at[pl.ds(r, K)], rows.at[0])
        pltpu.sync_copy(rows.at[0], out_hbm.at[pl.ds(c * NP + r, K)])
        return 0
    lax.fori_loop(0, NZ, xbody, 0)


def _bn_relu(z, g, be):
    m = jnp.mean(z, axis=0, keepdims=True)
    v = jnp.mean((z - m) ** 2, axis=0, keepdims=True)
    return jnp.maximum(g * (z - m) * lax.rsqrt(v + 1e-5) + be, 0.0)


NB = N // 8


def _segmax(p, bat, bat_ref, pscr_ref, pool_ref):
    pscr_ref[...] = p
    bm = jnp.max(p.reshape(NB, 8, CP), axis=1)
    lane = lax.broadcasted_iota(jnp.int32, (1, 2 * NG), 1)
    starts = jnp.sum((bat < lane).astype(jnp.float32), axis=0,
                     keepdims=True).astype(jnp.int32)
    biota = lax.broadcasted_iota(jnp.int32, (NB, 1), 0)

    def gbody(g, _):
        s_g = jnp.sum(jnp.where(lane == g, starts, 0))
        e_g = jnp.sum(jnp.where(lane == g + 1, starts, 0))
        b0 = jnp.clip(s_g // 8, 0, NB - 1)
        b1 = jnp.clip((e_g - 1) // 8, 0, NB - 1)
        inner = jnp.max(jnp.where((biota > b0) & (biota < b1), bm, -jnp.inf),
                        axis=0, keepdims=True)
        r0 = pscr_ref[pl.ds(b0 * 8, 8), :]
        m0 = bat_ref[pl.ds(b0 * 8, 8), :] == g
        v0 = jnp.max(jnp.where(m0, r0, -jnp.inf), axis=0, keepdims=True)
        r1 = pscr_ref[pl.ds(b1 * 8, 8), :]
        m1 = bat_ref[pl.ds(b1 * 8, 8), :] == g
        v1 = jnp.max(jnp.where(m1, r1, -jnp.inf), axis=0, keepdims=True)
        pool_ref[pl.ds(g, 1), :] = jnp.maximum(jnp.maximum(inner, v0), v1)
        return 0
    lax.fori_loop(0, NG, gbody, 0)


def _stage0_body(x_ref, w_ref, b_ref, g_ref, be_ref, wl_ref, bl_ref, bat_ref,
                 th_ref, h_ref, tab_ref, pool_ref, pscr_ref):
    z = jnp.dot(x_ref[...], w_ref[...], preferred_element_type=jnp.float32)
    h = _bn_relu(z + b_ref[...], g_ref[...], be_ref[...])
    h_ref[...] = h
    tab_ref[...] = th_ref[0] * h
    p = jnp.dot(h, wl_ref[...], preferred_element_type=jnp.float32) + bl_ref[...]
    _segmax(p, bat_ref[...], bat_ref, pscr_ref, pool_ref)


def _stage1_body(h_ref, acc_ref, w_ref, b_ref, g_ref, be_ref, wl_ref, bl_ref,
                 bat_ref, pin_ref, th_ref, h1_ref, tab_ref, pool_ref, pscr_ref):
    agg = acc_ref[pl.ds(0, N), :] + acc_ref[pl.ds(NP, N), :]
    z = jnp.dot(h_ref[...] + agg, w_ref[...], preferred_element_type=jnp.float32)
    h = _bn_relu(z + b_ref[...], g_ref[...], be_ref[...])
    h1_ref[...] = h
    tab_ref[pl.ds(0, N), :] = th_ref[0] * h
    tab_ref[pl.ds(N, N), :] = th_ref[1] * h
    p = jnp.dot(h, wl_ref[...], preferred_element_type=jnp.float32) + bl_ref[...]
    _segmax(p, bat_ref[...], bat_ref, pscr_ref, pool_ref)
    pool_ref[...] = pool_ref[...] + pin_ref[...]


def _stage2_body(h_ref, acc_ref, w_ref, b_ref, g_ref, be_ref, wl_ref, bl_ref,
                 bat_ref, pin_ref, pool_ref, pscr_ref):
    agg = acc_ref[pl.ds(0, N), :] + acc_ref[pl.ds(NP, N), :]
    z = jnp.dot(h_ref[...] + agg, w_ref[...], preferred_element_type=jnp.float32)
    h = _bn_relu(z + b_ref[...], g_ref[...], be_ref[...])
    p = jnp.dot(h, wl_ref[...], preferred_element_type=jnp.float32) + bl_ref[...]
    _segmax(p, bat_ref[...], bat_ref, pscr_ref, pool_ref)
    pool_ref[...] = pool_ref[...] + pin_ref[...]


def _vspec(n):
    return [pl.BlockSpec(memory_space=pltpu.VMEM) for _ in range(n)]


_SMEM = pl.BlockSpec(memory_space=pltpu.SMEM)


def kernel(x, edge_index, edge_weights, batch,
           W0, b0, g0, be0, Wl0, bl0,
           theta1, W1, b1, g1, be1, Wl1, bl1,
           theta2, W2, b2, g2, be2, Wl2, bl2):
    src = edge_index[0]
    dst = edge_index[1]
    ew = edge_weights

    spread = (jnp.arange(EPAD - E, dtype=jnp.int32) * 37) % N
    trash = TRASH + (jnp.arange(E, dtype=jnp.int32) % (NP - N))
    tpad = TRASH + ((jnp.arange(EPAD - E, dtype=jnp.int32) * 7) % (NP - N))
    gsrc1 = jnp.concatenate([src, spread]).reshape(NW * CH, K)
    sidx1 = jnp.concatenate([jnp.where(ew <= 0, dst, trash), tpad]).reshape(NW * CH, K)
    gsrc2 = jnp.concatenate([jnp.where(ew <= 1, ew * N + src, src), spread]).reshape(NW * CH, K)
    sidx2 = jnp.concatenate([jnp.where(ew <= 1, dst, trash), tpad]).reshape(NW * CH, K)
    zblk = jnp.zeros((K, H), jnp.float32)

    def row(v):
        return v.reshape(1, -1)
    def padwl(wl):
        return jnp.pad(wl, ((0, 0), (0, CP - C)))
    def padbl(bl):
        return jnp.pad(bl, (0, CP - C)).reshape(1, CP)
    bat = batch.reshape(N, 1)

    h0, tab0, p0 = pl.pallas_call(
        _stage0_body,
        in_specs=_vspec(8) + [_SMEM],
        out_shape=(jax.ShapeDtypeStruct((N, H), jnp.float32),
                   jax.ShapeDtypeStruct((N, H), jnp.float32),
                   jax.ShapeDtypeStruct((NG, CP), jnp.float32)),
        scratch_shapes=[pltpu.VMEM((N, CP), jnp.float32)],
    )(x, W0, row(b0), row(g0), row(be0), padwl(Wl0), padbl(bl0), bat, theta1)

    acc1 = _sc_segsum(tab0, gsrc1, sidx1, zblk)

    h1, tab1, p1 = pl.pallas_call(
        _stage1_body,
        in_specs=_vspec(10) + [_SMEM],
        out_shape=(jax.ShapeDtypeStruct((N, H), jnp.float32),
                   jax.ShapeDtypeStruct((2 * N, H), jnp.float32),
                   jax.ShapeDtypeStruct((NG, CP), jnp.float32)),
        scratch_shapes=[pltpu.VMEM((N, CP), jnp.float32)],
    )(h0, acc1, W1, row(b1), row(g1), row(be1), padwl(Wl1), padbl(bl1),
      bat, p0, theta2)

    acc2 = _sc_segsum(tab1, gsrc2, sidx2, zblk)

    p2, = pl.pallas_call(
        _stage2_body,
        in_specs=_vspec(10),
        out_shape=(jax.ShapeDtypeStruct((NG, CP), jnp.float32),),
        scratch_shapes=[pltpu.VMEM((N, CP), jnp.float32)],
    )(h1, acc2, W2, row(b2), row(g2), row(be2), padwl(Wl2), padbl(bl2),
      bat, p1)

    return p2[:, :C]
